# baseline (speedup 1.0000x reference)
import functools

import jax
import jax.numpy as jnp
from jax import lax
from jax.experimental import pallas as pl
from jax.experimental.pallas import tpu as pltpu
from jax.experimental.pallas import tpu_sc as plsc

NNODES = 10000
NEDGE = 320000
NREL = 3
FDIM = 64

NROW = 10240
NC, NS = 2, 16
NW = NC * NS
CH = 128
NBUF = 3
DEG_NBUF = 4
EPW = NEDGE // NW
NCH = 81
EPWP = NCH * CH
RPS = NROW // NS

DEG_DUMMY = 6 * NROW
ACC6 = 63488
DSL = ACC6 // NS
DPW = 6 * NEDGE // NW
DCH = 472
DPWP = DCH * CH

BLK = 2048
GRID = NROW // BLK

def _mesh():
    return plsc.VectorSubcoreMesh(
        core_axis_name="c", subcore_axis_name="s",
        num_cores=NC, num_subcores=NS)


@functools.cache
def _deg_kernel():
    return functools.partial(
        pl.kernel,
        out_type=jax.ShapeDtypeStruct((NC, 1, ACC6), jnp.float32),
        mesh=_mesh(),
        scratch_types=[
            pltpu.VMEM((DCH, 1, CH), jnp.int32),
            pltpu.VMEM((CH,), jnp.float32),
            pltpu.VMEM((DSL,), jnp.float32),
            pltpu.VMEM_SHARED((ACC6,), jnp.float32),
        ] + [pltpu.SemaphoreType.DMA] * DEG_NBUF,
    )(_deg_body)


def _deg_body(idx_hbm, out_hbm, ev, onesv, zbuf, acc, *ssem):
    cid = lax.axis_index("c")
    sid = lax.axis_index("s")
    wid = cid * NS + sid
    one16 = jnp.full((16,), 1.0, jnp.float32)
    zero16 = jnp.zeros((16,), jnp.float32)

    def initb(i, _):
        onesv[pl.ds(i * 16, 16)] = one16
        return 0

    lax.fori_loop(0, CH // 16, initb, 0)

    def initz(i, _):
        zbuf[pl.ds(i * 16, 16)] = zero16
        return 0

    lax.fori_loop(0, DSL // 16, initz, 0)
    pltpu.sync_copy(idx_hbm.at[wid], ev)
    pltpu.sync_copy(zbuf, acc.at[pl.ds(sid * DSL, DSL)])
    plsc.subcore_barrier()

    def group(g, _):
        for j in range(DEG_NBUF):
            d = g * DEG_NBUF + j

            @pl.when(d >= DEG_NBUF)
            def _():
                pltpu.make_async_copy(
                    onesv, acc.at[ev.at[d - DEG_NBUF, 0]], ssem[j]).wait()

            pltpu.async_copy(onesv, acc.at[ev.at[d, 0]], ssem[j], add=True)
        return 0

    lax.fori_loop(0, DCH // DEG_NBUF, group, 0)
    for j in range(DEG_NBUF):
        pltpu.make_async_copy(
            onesv, acc.at[ev.at[DCH - DEG_NBUF + j, 0]], ssem[j]).wait()
    plsc.subcore_barrier()
    pltpu.sync_copy(acc.at[pl.ds(sid * DSL, DSL)],
                    out_hbm.at[cid, 0, pl.ds(sid * DSL, DSL)])


@functools.cache
def _agg_kernel():
    return functools.partial(
        pl.kernel,
        out_type=jax.ShapeDtypeStruct((NC, NREL, NROW, FDIM), jnp.float32),
        mesh=_mesh(),
        scratch_types=[
            pltpu.VMEM((NCH, 2, CH), jnp.int32),
            pltpu.VMEM((NBUF, CH, FDIM), jnp.float32),
            pltpu.VMEM_SHARED((NROW, FDIM), jnp.float32),
            pltpu.VMEM_SHARED((NROW, FDIM), jnp.float32),
        ] + [pltpu.SemaphoreType.DMA] * (2 * NBUF),
        compiler_params=pltpu.CompilerParams(use_tc_tiling_on_sc=False),
    )(_agg_body)


def _agg_body(tab_hbm, edges_hbm, out_hbm, ev, rows, tab_sp, acc, *sems):
    gsem = sems[:NBUF]
    ssem = sems[NBUF:]
    cid = lax.axis_index("c")
    sid = lax.axis_index("s")
    wid = cid * NS + sid
    zero16 = jnp.zeros((16,), jnp.float32)
    stripe = pl.ds(sid * RPS, RPS)

    for r in range(NREL):
        tab_copy = pltpu.async_copy(tab_hbm.at[r].at[stripe],
                                    tab_sp.at[stripe], ssem[0])
        ev_copy = pltpu.async_copy(edges_hbm.at[r, wid], ev, gsem[NBUF - 1])

        def initz(i, _):
            rows[0, i // 4, pl.ds((i % 4) * 16, 16)] = zero16
            return 0

        lax.fori_loop(0, CH * 4, initz, 0)
        for k in range(RPS // CH):
            pltpu.sync_copy(rows.at[0], acc.at[pl.ds(sid * RPS + k * CH, CH)])
        tab_copy.wait()
        ev_copy.wait()
        plsc.subcore_barrier()

        for j in range(NBUF - 1):
            pltpu.async_copy(tab_sp.at[ev.at[j, 0]], rows.at[j], gsem[j])

        def group(g, _):
            for j in range(NBUF):
                d = g * NBUF + j
                pltpu.make_async_copy(
                    tab_sp.at[ev.at[d, 0]], rows.at[j], gsem[j]).wait()
                pltpu.async_copy(rows.at[j], acc.at[ev.at[d, 1]], ssem[j],
                                 add=True)
                jn = (j + NBUF - 1) % NBUF

                @pl.when(d + NBUF - 1 < NCH)
                def _():
                    @pl.when(d >= 1)
                    def _():
                        pltpu.make_async_copy(
                            rows.at[jn], acc.at[ev.at[d - 1, 1]],
                            ssem[jn]).wait()

                    pltpu.async_copy(tab_sp.at[ev.at[d + NBUF - 1, 0]],
                                     rows.at[jn], gsem[jn])
            return 0

        lax.fori_loop(0, NCH // NBUF, group, 0)
        for j in range(NBUF):
            pltpu.make_async_copy(
                rows.at[j], acc.at[ev.at[NCH - NBUF + j, 1]], ssem[j]).wait()
        plsc.subcore_barrier()
        pltpu.sync_copy(acc.at[stripe], out_hbm.at[cid, r, stripe])
        plsc.subcore_barrier()


def _elu(v):
    return jnp.where(v > 0, v, jnp.exp(v) - 1.0)


def _encpre_body(x_ref, n_ref, w1_ref, b1_ref, w2_ref, b2_ref, dsrc_ref,
                 h_ref):
    a = x_ref[...] + n_ref[...]
    a = _elu(jnp.dot(a, w1_ref[...], preferred_element_type=jnp.float32,
                     precision=lax.Precision.HIGHEST) + b1_ref[...])
    f = _elu(jnp.dot(a, w2_ref[...], preferred_element_type=jnp.float32,
                     precision=lax.Precision.HIGHEST) + b2_ref[...])
    deg = dsrc_ref[0] + dsrc_ref[1]
    ns = jnp.where(deg > 0, lax.rsqrt(deg), 0.0)
    h_ref[...] = f[None, :, :] * ns[:, :, None]


def _run_encpre(xp, noisep, w1, b1, w2, b2, dsrc):
    return pl.pallas_call(
        _encpre_body,
        grid=(GRID,),
        in_specs=[
            pl.BlockSpec((BLK, 128), lambda i: (i, 0)),
            pl.BlockSpec((BLK, 128), lambda i: (i, 0)),
            pl.BlockSpec((128, 96), lambda i: (0, 0)),
            pl.BlockSpec((1, 96), lambda i: (0, 0)),
            pl.BlockSpec((96, 64), lambda i: (0, 0)),
            pl.BlockSpec((1, 64), lambda i: (0, 0)),
            pl.BlockSpec((NC, NREL, BLK), lambda i: (0, 0, i)),
        ],
        out_specs=pl.BlockSpec((NREL, BLK, FDIM), lambda i: (0, i, 0)),
        out_shape=jax.ShapeDtypeStruct((NREL, NROW, FDIM), jnp.float32),
    )(xp, noisep, w1, b1, w2, b2, dsrc)


def _combine(p_ref, ddst_ref, wg_ref, bg_ref, g_ref, be_ref, ws1_ref, ws2_ref):
    din = ddst_ref[0] + ddst_ref[1]
    nd = jnp.where(din > 0, lax.rsqrt(din), 0.0)
    outs = []
    for r in range(NREL):
        agg = p_ref[0, r] + p_ref[1, r]
        t = jnp.dot(agg, wg_ref[...], preferred_element_type=jnp.float32,
                 precision=lax.Precision.HIGHEST)
        outs.append(_elu(t * nd[r][:, None] + bg_ref[...]))
    hs = jnp.stack(outs, axis=0)
    mu = jnp.mean(hs, axis=-1, keepdims=True)
    var = jnp.mean((hs - mu) ** 2, axis=-1, keepdims=True)
    ln = (hs - mu) * lax.rsqrt(var + 1e-5) * g_ref[...] + be_ref[...]
    logits = []
    for r in range(NREL):
        s = jax.nn.sigmoid(jnp.dot(ln[r], ws1_ref[r],
                                   preferred_element_type=jnp.float32,
                 precision=lax.Precision.HIGHEST))
        logits.append(jnp.sum(s * ws2_ref[r][None, :], axis=-1))
    lg = jnp.stack(logits, axis=0)
    e = jnp.exp(lg - jnp.max(lg, axis=0, keepdims=True))
    att = e / jnp.sum(e, axis=0, keepdims=True)
    return jnp.sum(att[:, :, None] * ln, axis=0)


def _post0_body(p_ref, ddst_ref, dsrc_ref, wg_ref, bg_ref, g_ref, be_ref,
                ws1_ref, ws2_ref, hall_ref):
    h = _combine(p_ref, ddst_ref, wg_ref, bg_ref, g_ref, be_ref, ws1_ref,
                 ws2_ref)
    deg = dsrc_ref[0] + dsrc_ref[1]
    ns = jnp.where(deg > 0, lax.rsqrt(deg), 0.0)
    hall_ref[...] = h[None, :, :] * ns[:, :, None]


def _run_post0(p, ddst, dsrc, wg, bg, g, be, ws1, ws2):
    return pl.pallas_call(
        _post0_body,
        grid=(GRID,),
        in_specs=[
            pl.BlockSpec((NC, NREL, BLK, FDIM), lambda i: (0, 0, i, 0)),
            pl.BlockSpec((NC, NREL, BLK), lambda i: (0, 0, i)),
            pl.BlockSpec((NC, NREL, BLK), lambda i: (0, 0, i)),
            pl.BlockSpec((FDIM, FDIM), lambda i: (0, 0)),
            pl.BlockSpec((1, FDIM), lambda i: (0, 0)),
            pl.BlockSpec((1, FDIM), lambda i: (0, 0)),
            pl.BlockSpec((1, FDIM), lambda i: (0, 0)),
            pl.BlockSpec((NREL, FDIM, FDIM), lambda i: (0, 0, 0)),
            pl.BlockSpec((NREL, FDIM), lambda i: (0, 0)),
        ],
        out_specs=pl.BlockSpec((NREL, BLK, FDIM), lambda i: (0, i, 0)),
        out_shape=jax.ShapeDtypeStruct((NREL, NROW, FDIM), jnp.float32),
    )(p, ddst, dsrc, wg, bg, g, be, ws1, ws2)


def _post1_body(p_ref, ddst_ref, wg_ref, bg_ref, g_ref, be_ref, ws1_ref,
                ws2_ref, wh_ref, bh_ref, out_ref):
    h = _combine(p_ref, ddst_ref, wg_ref, bg_ref, g_ref, be_ref, ws1_ref,
                 ws2_ref)
    out_ref[...] = (jnp.sum(h * wh_ref[...], axis=-1, keepdims=True)
                    + bh_ref[...])


def _run_post1(p, ddst, wg, bg, g, be, ws1, ws2, wh, bh):
    return pl.pallas_call(
        _post1_body,
        grid=(GRID,),
        in_specs=[
            pl.BlockSpec((NC, NREL, BLK, FDIM), lambda i: (0, 0, i, 0)),
            pl.BlockSpec((NC, NREL, BLK), lambda i: (0, 0, i)),
            pl.BlockSpec((FDIM, FDIM), lambda i: (0, 0)),
            pl.BlockSpec((1, FDIM), lambda i: (0, 0)),
            pl.BlockSpec((1, FDIM), lambda i: (0, 0)),
            pl.BlockSpec((1, FDIM), lambda i: (0, 0)),
            pl.BlockSpec((NREL, FDIM, FDIM), lambda i: (0, 0, 0)),
            pl.BlockSpec((NREL, FDIM), lambda i: (0, 0)),
            pl.BlockSpec((1, FDIM), lambda i: (0, 0)),
            pl.BlockSpec((1, 1), lambda i: (0, 0)),
        ],
        out_specs=pl.BlockSpec((BLK, 1), lambda i: (i, 0)),
        out_shape=jax.ShapeDtypeStruct((NROW, 1), jnp.float32),
    )(p, ddst, wg, bg, g, be, ws1, ws2, wh, bh)


def kernel(x, edge_index, enc_W1, enc_b1, enc_W2, enc_b2, Wg0, bg0, g0, be0,
           Ws1_0, Ws2_0, Wg1, bg1, g1, be1, Ws1_1, Ws2_1, Wh, bh):
    noise = jax.random.normal(jax.random.key(42), x.shape, dtype=x.dtype)
    pad = NROW - NNODES
    xp = jnp.pad(x, ((0, pad), (0, 0)))
    noisep = jnp.pad(noise, ((0, pad), (0, 0)))
    ei = edge_index.astype(jnp.int32)

    flat6 = (ei.reshape(6, NEDGE)
             + (jnp.arange(6, dtype=jnp.int32) * NROW)[:, None])
    flat6 = flat6.reshape(NW, DPW)
    dpad = jnp.full((NW, DPWP - DPW), DEG_DUMMY, jnp.int32)
    didx = jnp.concatenate([flat6, dpad], axis=1).reshape(NW, DCH, 1, CH)

    s_ = ei[:, 0, :].reshape(NREL, NW, EPW)
    d_ = ei[:, 1, :].reshape(NREL, NW, EPW)
    epad = jnp.full((NREL, NW, EPWP - EPW), NNODES, jnp.int32)
    s_ = jnp.concatenate([s_, epad], axis=2).reshape(NREL, NW, NCH, CH)
    d_ = jnp.concatenate([d_, epad], axis=2).reshape(NREL, NW, NCH, CH)
    edges = jnp.stack([s_, d_], axis=3)

    degp = _deg_kernel()(didx)
    d6 = degp.reshape(NC, ACC6)[:, :DEG_DUMMY].reshape(NC, 6, NROW)
    dsrc = d6[:, 0::2, :]
    ddst = d6[:, 1::2, :]

    h0 = _run_encpre(xp, noisep, enc_W1, enc_b1.reshape(1, -1), enc_W2,
                     enc_b2.reshape(1, -1), dsrc)
    p0 = _agg_kernel()(h0, edges)
    h1 = _run_post0(p0, ddst, dsrc, Wg0, bg0.reshape(1, -1), g0.reshape(1, -1),
                    be0.reshape(1, -1), Ws1_0, Ws2_0[..., 0])
    p1 = _agg_kernel()(h1, edges)
    out = _run_post1(p1, ddst, Wg1, bg1.reshape(1, -1), g1.reshape(1, -1),
                     be1.reshape(1, -1), Ws1_1, Ws2_1[..., 0],
                     Wh[:, 0].reshape(1, -1), bh.reshape(1, 1))
    return out[:NNODES]


# speedup vs baseline: 1.0075x; 1.0075x over previous
import functools

import jax
import jax.numpy as jnp
from jax import lax
from jax.experimental import pallas as pl
from jax.experimental.pallas import tpu as pltpu
from jax.experimental.pallas import tpu_sc as plsc

NNODES = 10000
NEDGE = 320000
NREL = 3
FDIM = 64

NROW = 10240
NC, NS = 2, 16
NW = NC * NS
CH = 128
NBUF = 3
DEG_NBUF = 4
EPW = NEDGE // NW
NCH = 81
EPWP = NCH * CH
RPS = NROW // NS

DEG_DUMMY = 6 * NROW
ACC6 = 63488
DSL = ACC6 // NS
DPW = 6 * NEDGE // NW
DCH = 472
DPWP = DCH * CH

BLK = 1024
GRID = NROW // BLK

def _mesh():
    return plsc.VectorSubcoreMesh(
        core_axis_name="c", subcore_axis_name="s",
        num_cores=NC, num_subcores=NS)


@functools.cache
def _deg_kernel():
    return functools.partial(
        pl.kernel,
        out_type=jax.ShapeDtypeStruct((NC, 1, ACC6), jnp.float32),
        mesh=_mesh(),
        scratch_types=[
            pltpu.VMEM((DCH, 1, CH), jnp.int32),
            pltpu.VMEM((CH,), jnp.float32),
            pltpu.VMEM((DSL,), jnp.float32),
            pltpu.VMEM_SHARED((ACC6,), jnp.float32),
        ] + [pltpu.SemaphoreType.DMA] * DEG_NBUF,
    )(_deg_body)


def _deg_body(idx_hbm, out_hbm, ev, onesv, zbuf, acc, *ssem):
    cid = lax.axis_index("c")
    sid = lax.axis_index("s")
    wid = cid * NS + sid
    one16 = jnp.full((16,), 1.0, jnp.float32)
    zero16 = jnp.zeros((16,), jnp.float32)
    ev_copy = pltpu.async_copy(idx_hbm.at[wid], ev, ssem[0])

    def initb(i, _):
        onesv[pl.ds(i * 16, 16)] = one16
        return 0

    lax.fori_loop(0, CH // 16, initb, 0)

    def initz(i, _):
        zbuf[pl.ds(i * 16, 16)] = zero16
        return 0

    lax.fori_loop(0, DSL // 16, initz, 0)
    pltpu.sync_copy(zbuf, acc.at[pl.ds(sid * DSL, DSL)])
    ev_copy.wait()
    plsc.subcore_barrier()

    def group(g, _):
        for j in range(DEG_NBUF):
            d = g * DEG_NBUF + j

            @pl.when(d >= DEG_NBUF)
            def _():
                pltpu.make_async_copy(
                    onesv, acc.at[ev.at[d - DEG_NBUF, 0]], ssem[j]).wait()

            pltpu.async_copy(onesv, acc.at[ev.at[d, 0]], ssem[j], add=True)
        return 0

    lax.fori_loop(0, DCH // DEG_NBUF, group, 0)
    for j in range(DEG_NBUF):
        pltpu.make_async_copy(
            onesv, acc.at[ev.at[DCH - DEG_NBUF + j, 0]], ssem[j]).wait()
    plsc.subcore_barrier()
    pltpu.sync_copy(acc.at[pl.ds(sid * DSL, DSL)],
                    out_hbm.at[cid, 0, pl.ds(sid * DSL, DSL)])


@functools.cache
def _agg_kernel():
    return functools.partial(
        pl.kernel,
        out_type=jax.ShapeDtypeStruct((NC, NREL, NROW, FDIM), jnp.float32),
        mesh=_mesh(),
        scratch_types=[
            pltpu.VMEM((NCH, 2, CH), jnp.int32),
            pltpu.VMEM((NBUF, CH, FDIM), jnp.float32),
            pltpu.VMEM_SHARED((NROW, FDIM), jnp.float32),
            pltpu.VMEM_SHARED((NROW, FDIM), jnp.float32),
        ] + [pltpu.SemaphoreType.DMA] * (2 * NBUF),
        compiler_params=pltpu.CompilerParams(use_tc_tiling_on_sc=False),
    )(_agg_body)


def _agg_body(tab_hbm, edges_hbm, out_hbm, ev, rows, tab_sp, acc, *sems):
    gsem = sems[:NBUF]
    ssem = sems[NBUF:]
    cid = lax.axis_index("c")
    sid = lax.axis_index("s")
    wid = cid * NS + sid
    zero16 = jnp.zeros((16,), jnp.float32)
    stripe = pl.ds(sid * RPS, RPS)

    for r in range(NREL):
        tab_copy = pltpu.async_copy(tab_hbm.at[r].at[stripe],
                                    tab_sp.at[stripe], ssem[0])
        ev_copy = pltpu.async_copy(edges_hbm.at[r, wid], ev, gsem[NBUF - 1])

        def initz(i, _):
            rows[0, i // 4, pl.ds((i % 4) * 16, 16)] = zero16
            return 0

        lax.fori_loop(0, CH * 4, initz, 0)
        for k in range(RPS // CH):
            pltpu.sync_copy(rows.at[0], acc.at[pl.ds(sid * RPS + k * CH, CH)])
        tab_copy.wait()
        ev_copy.wait()
        plsc.subcore_barrier()

        for j in range(NBUF - 1):
            pltpu.async_copy(tab_sp.at[ev.at[j, 0]], rows.at[j], gsem[j])

        def group(g, _):
            for j in range(NBUF):
                d = g * NBUF + j
                pltpu.make_async_copy(
                    tab_sp.at[ev.at[d, 0]], rows.at[j], gsem[j]).wait()
                pltpu.async_copy(rows.at[j], acc.at[ev.at[d, 1]], ssem[j],
                                 add=True)
                jn = (j + NBUF - 1) % NBUF

                @pl.when(d + NBUF - 1 < NCH)
                def _():
                    @pl.when(d >= 1)
                    def _():
                        pltpu.make_async_copy(
                            rows.at[jn], acc.at[ev.at[d - 1, 1]],
                            ssem[jn]).wait()

                    pltpu.async_copy(tab_sp.at[ev.at[d + NBUF - 1, 0]],
                                     rows.at[jn], gsem[jn])
            return 0

        lax.fori_loop(0, NCH // NBUF, group, 0)
        for j in range(NBUF):
            pltpu.make_async_copy(
                rows.at[j], acc.at[ev.at[NCH - NBUF + j, 1]], ssem[j]).wait()
        plsc.subcore_barrier()
        pltpu.sync_copy(acc.at[stripe], out_hbm.at[cid, r, stripe])
        plsc.subcore_barrier()


def _elu(v):
    return jnp.where(v > 0, v, jnp.exp(v) - 1.0)


def _encpre_body(x_ref, n_ref, w1_ref, b1_ref, w2_ref, b2_ref, dsrc_ref,
                 h_ref):
    a = x_ref[...] + n_ref[...]
    a = _elu(jnp.dot(a, w1_ref[...], preferred_element_type=jnp.float32,
                     precision=lax.Precision.HIGHEST) + b1_ref[...])
    f = _elu(jnp.dot(a, w2_ref[...], preferred_element_type=jnp.float32,
                     precision=lax.Precision.HIGHEST) + b2_ref[...])
    deg = dsrc_ref[0] + dsrc_ref[1]
    ns = jnp.where(deg > 0, lax.rsqrt(deg), 0.0)
    h_ref[...] = f[None, :, :] * ns[:, :, None]


def _run_encpre(xp, noisep, w1, b1, w2, b2, dsrc):
    return pl.pallas_call(
        _encpre_body,
        grid=(GRID,),
        in_specs=[
            pl.BlockSpec((BLK, 128), lambda i: (i, 0)),
            pl.BlockSpec((BLK, 128), lambda i: (i, 0)),
            pl.BlockSpec((128, 96), lambda i: (0, 0)),
            pl.BlockSpec((1, 96), lambda i: (0, 0)),
            pl.BlockSpec((96, 64), lambda i: (0, 0)),
            pl.BlockSpec((1, 64), lambda i: (0, 0)),
            pl.BlockSpec((NC, NREL, BLK), lambda i: (0, 0, i)),
        ],
        out_specs=pl.BlockSpec((NREL, BLK, FDIM), lambda i: (0, i, 0)),
        out_shape=jax.ShapeDtypeStruct((NREL, NROW, FDIM), jnp.float32),
    )(xp, noisep, w1, b1, w2, b2, dsrc)


def _combine(p_ref, ddst_ref, wg_ref, bg_ref, g_ref, be_ref, ws1_ref, ws2_ref):
    din = ddst_ref[0] + ddst_ref[1]
    nd = jnp.where(din > 0, lax.rsqrt(din), 0.0)
    outs = []
    for r in range(NREL):
        agg = p_ref[0, r] + p_ref[1, r]
        t = jnp.dot(agg, wg_ref[...], preferred_element_type=jnp.float32,
                 precision=lax.Precision.HIGHEST)
        outs.append(_elu(t * nd[r][:, None] + bg_ref[...]))
    hs = jnp.stack(outs, axis=0)
    mu = jnp.mean(hs, axis=-1, keepdims=True)
    var = jnp.mean((hs - mu) ** 2, axis=-1, keepdims=True)
    ln = (hs - mu) * lax.rsqrt(var + 1e-5) * g_ref[...] + be_ref[...]
    logits = []
    for r in range(NREL):
        s = jax.nn.sigmoid(jnp.dot(ln[r], ws1_ref[r],
                                   preferred_element_type=jnp.float32,
                 precision=lax.Precision.HIGHEST))
        logits.append(jnp.sum(s * ws2_ref[r][None, :], axis=-1))
    lg = jnp.stack(logits, axis=0)
    e = jnp.exp(lg - jnp.max(lg, axis=0, keepdims=True))
    att = e / jnp.sum(e, axis=0, keepdims=True)
    return jnp.sum(att[:, :, None] * ln, axis=0)


def _post0_body(p_ref, ddst_ref, dsrc_ref, wg_ref, bg_ref, g_ref, be_ref,
                ws1_ref, ws2_ref, hall_ref):
    h = _combine(p_ref, ddst_ref, wg_ref, bg_ref, g_ref, be_ref, ws1_ref,
                 ws2_ref)
    deg = dsrc_ref[0] + dsrc_ref[1]
    ns = jnp.where(deg > 0, lax.rsqrt(deg), 0.0)
    hall_ref[...] = h[None, :, :] * ns[:, :, None]


def _run_post0(p, ddst, dsrc, wg, bg, g, be, ws1, ws2):
    return pl.pallas_call(
        _post0_body,
        grid=(GRID,),
        in_specs=[
            pl.BlockSpec((NC, NREL, BLK, FDIM), lambda i: (0, 0, i, 0)),
            pl.BlockSpec((NC, NREL, BLK), lambda i: (0, 0, i)),
            pl.BlockSpec((NC, NREL, BLK), lambda i: (0, 0, i)),
            pl.BlockSpec((FDIM, FDIM), lambda i: (0, 0)),
            pl.BlockSpec((1, FDIM), lambda i: (0, 0)),
            pl.BlockSpec((1, FDIM), lambda i: (0, 0)),
            pl.BlockSpec((1, FDIM), lambda i: (0, 0)),
            pl.BlockSpec((NREL, FDIM, FDIM), lambda i: (0, 0, 0)),
            pl.BlockSpec((NREL, FDIM), lambda i: (0, 0)),
        ],
        out_specs=pl.BlockSpec((NREL, BLK, FDIM), lambda i: (0, i, 0)),
        out_shape=jax.ShapeDtypeStruct((NREL, NROW, FDIM), jnp.float32),
    )(p, ddst, dsrc, wg, bg, g, be, ws1, ws2)


def _post1_body(p_ref, ddst_ref, wg_ref, bg_ref, g_ref, be_ref, ws1_ref,
                ws2_ref, wh_ref, bh_ref, out_ref):
    h = _combine(p_ref, ddst_ref, wg_ref, bg_ref, g_ref, be_ref, ws1_ref,
                 ws2_ref)
    out_ref[...] = (jnp.sum(h * wh_ref[...], axis=-1, keepdims=True)
                    + bh_ref[...])


def _run_post1(p, ddst, wg, bg, g, be, ws1, ws2, wh, bh):
    return pl.pallas_call(
        _post1_body,
        grid=(GRID,),
        in_specs=[
            pl.BlockSpec((NC, NREL, BLK, FDIM), lambda i: (0, 0, i, 0)),
            pl.BlockSpec((NC, NREL, BLK), lambda i: (0, 0, i)),
            pl.BlockSpec((FDIM, FDIM), lambda i: (0, 0)),
            pl.BlockSpec((1, FDIM), lambda i: (0, 0)),
            pl.BlockSpec((1, FDIM), lambda i: (0, 0)),
            pl.BlockSpec((1, FDIM), lambda i: (0, 0)),
            pl.BlockSpec((NREL, FDIM, FDIM), lambda i: (0, 0, 0)),
            pl.BlockSpec((NREL, FDIM), lambda i: (0, 0)),
            pl.BlockSpec((1, FDIM), lambda i: (0, 0)),
            pl.BlockSpec((1, 1), lambda i: (0, 0)),
        ],
        out_specs=pl.BlockSpec((BLK, 1), lambda i: (i, 0)),
        out_shape=jax.ShapeDtypeStruct((NROW, 1), jnp.float32),
    )(p, ddst, wg, bg, g, be, ws1, ws2, wh, bh)


def kernel(x, edge_index, enc_W1, enc_b1, enc_W2, enc_b2, Wg0, bg0, g0, be0,
           Ws1_0, Ws2_0, Wg1, bg1, g1, be1, Ws1_1, Ws2_1, Wh, bh):
    noise = jax.random.normal(jax.random.key(42), x.shape, dtype=x.dtype)
    pad = NROW - NNODES
    xp = jnp.pad(x, ((0, pad), (0, 0)))
    noisep = jnp.pad(noise, ((0, pad), (0, 0)))
    ei = edge_index.astype(jnp.int32)

    flat6 = (ei.reshape(6, NEDGE)
             + (jnp.arange(6, dtype=jnp.int32) * NROW)[:, None])
    flat6 = flat6.reshape(NW, DPW)
    dpad = jnp.full((NW, DPWP - DPW), DEG_DUMMY, jnp.int32)
    didx = jnp.concatenate([flat6, dpad], axis=1).reshape(NW, DCH, 1, CH)

    s_ = ei[:, 0, :].reshape(NREL, NW, EPW)
    d_ = ei[:, 1, :].reshape(NREL, NW, EPW)
    epad = jnp.full((NREL, NW, EPWP - EPW), NNODES, jnp.int32)
    s_ = jnp.concatenate([s_, epad], axis=2).reshape(NREL, NW, NCH, CH)
    d_ = jnp.concatenate([d_, epad], axis=2).reshape(NREL, NW, NCH, CH)
    edges = jnp.stack([s_, d_], axis=3)

    degp = _deg_kernel()(didx)
    d6 = degp.reshape(NC, ACC6)[:, :DEG_DUMMY].reshape(NC, 6, NROW)
    dsrc = d6[:, 0::2, :]
    ddst = d6[:, 1::2, :]

    h0 = _run_encpre(xp, noisep, enc_W1, enc_b1.reshape(1, -1), enc_W2,
                     enc_b2.reshape(1, -1), dsrc)
    p0 = _agg_kernel()(h0, edges)
    h1 = _run_post0(p0, ddst, dsrc, Wg0, bg0.reshape(1, -1), g0.reshape(1, -1),
                    be0.reshape(1, -1), Ws1_0, Ws2_0[..., 0])
    p1 = _agg_kernel()(h1, edges)
    out = _run_post1(p1, ddst, Wg1, bg1.reshape(1, -1), g1.reshape(1, -1),
                     be1.reshape(1, -1), Ws1_1, Ws2_1[..., 0],
                     Wh[:, 0].reshape(1, -1), bh.reshape(1, 1))
    return out[:NNODES]
